# baseline (speedup 1.0000x reference)
import functools

import jax
import jax.numpy as jnp
from jax import lax
from jax.experimental import pallas as pl
from jax.experimental.pallas import tpu as pltpu
from jax.experimental.pallas import tpu_sc as plsc

B = 64
L = 512
D = 768
NLANE = 16
NCHUNK = D // NLANE
NC = 2
NS = 16
STRIP = L // (NC * NS)
EPS = 1e-5


_PIB = lax.GatherScatterMode.PROMISE_IN_BOUNDS


_GDN = lax.GatherDimensionNumbers(
    offset_dims=(), collapsed_slice_dims=(0,), start_index_map=(0,))


def _gather16(v, idx):
    return lax.gather(v, idx[:, None], _GDN, slice_sizes=(1,), mode=_PIB)


def _lanesum(v, lane):
    for sh in (8, 4, 2, 1):
        v = v + _gather16(v, lane ^ sh)
    return v


def _rsqrt_vec(v):
    i = lax.bitcast_convert_type(v, jnp.int32)
    i = jnp.int32(0x5F3759DF) - lax.shift_right_arithmetic(i, 1)
    y = lax.bitcast_convert_type(i, jnp.float32)
    half = v * 0.5
    for _ in range(3):
        y = y * (1.5 - half * y * y)
    return y


def _bert_embed_sc(x_hbm, seg_hbm, tok_hbm, pos_hbm, segt_hbm, gamma_hbm,
                   beta_hbm, out_hbm, xb, sb, posb, segtb, gmb, btb, comb,
                   rows, gsem):
    w = lax.axis_index("s") * NC + lax.axis_index("c")
    p0 = w * STRIP

    pltpu.sync_copy(pos_hbm.at[pl.ds(p0, STRIP), :], posb)
    pltpu.sync_copy(segt_hbm, segtb)
    pltpu.sync_copy(gamma_hbm, gmb)
    pltpu.sync_copy(beta_hbm, btb)

    def build_comb(p, _):
        def build_c(c, _):
            off = c * NLANE
            pv = posb[p, pl.ds(off, NLANE)]
            for s in range(3):
                comb[s * STRIP + p, pl.ds(off, NLANE)] = (
                    pv + segtb[s, pl.ds(off, NLANE)])
            return 0
        lax.fori_loop(0, NCHUNK, build_c, 0)
        return 0
    lax.fori_loop(0, STRIP, build_comb, 0)

    lane = lax.iota(jnp.int32, NLANE)

    def seq_body(b, _):
        pltpu.sync_copy(x_hbm.at[pl.ds(b * L + p0, STRIP)], xb)
        pltpu.sync_copy(seg_hbm.at[pl.ds(b * L + p0, STRIP)], sb)
        pltpu.async_copy(tok_hbm.at[xb], rows, gsem).wait()

        sv = sb[...]

        def row_body(r, _):
            rot = _gather16(sv, (lane + r) & (NLANE - 1))
            q = rot[0] * STRIP + r
            acc = jnp.zeros((NLANE,), jnp.float32)
            acc2 = jnp.zeros((NLANE,), jnp.float32)
            for c in range(NCHUNK):
                off = c * NLANE
                t = rows[r, pl.ds(off, NLANE)] + comb[q, pl.ds(off, NLANE)]
                acc = acc + t
                acc2 = acc2 + t * t
                rows[r, pl.ds(off, NLANE)] = t
            mvec = _lanesum(acc, lane) * (1.0 / D)
            var = _lanesum(acc2, lane) * (1.0 / D) - mvec * mvec
            avec = _rsqrt_vec(var + EPS)
            for c in range(NCHUNK):
                off = c * NLANE
                t = rows[r, pl.ds(off, NLANE)]
                y = (t - mvec) * avec
                rows[r, pl.ds(off, NLANE)] = (
                    y * gmb[pl.ds(off, NLANE)] + btb[pl.ds(off, NLANE)])
            return 0
        lax.fori_loop(0, STRIP, row_body, 0)

        pltpu.sync_copy(rows, out_hbm.at[b, pl.ds(p0, STRIP), :])
        return 0
    lax.fori_loop(0, B, seq_body, 0)


@jax.jit
def _run(x, segment_label, token_table, pos_table, seg_table, gamma, beta):
    mesh = plsc.VectorSubcoreMesh(core_axis_name="c", subcore_axis_name="s")
    k = functools.partial(
        pl.kernel,
        out_type=jax.ShapeDtypeStruct((B, L, D), jnp.float32),
        mesh=mesh,
        scratch_types=[
            pltpu.VMEM((STRIP,), jnp.int32),
            pltpu.VMEM((STRIP,), jnp.int32),
            pltpu.VMEM((STRIP, D), jnp.float32),
            pltpu.VMEM((3, D), jnp.float32),
            pltpu.VMEM((D,), jnp.float32),
            pltpu.VMEM((D,), jnp.float32),
            pltpu.VMEM((3 * STRIP, D), jnp.float32),
            pltpu.VMEM((STRIP, D), jnp.float32),
            pltpu.SemaphoreType.DMA,
        ],
    )(_bert_embed_sc)
    return k(x, segment_label, token_table, pos_table, seg_table, gamma, beta)


def kernel(x, segment_label, token_table, pos_table, seg_table, gamma, beta):
    return _run(x.astype(jnp.int32).reshape(B * L),
                segment_label.astype(jnp.int32).reshape(B * L),
                token_table, pos_table, seg_table, gamma, beta)
